# baseline (speedup 1.0000x reference)
import jax
import jax.numpy as jnp
from jax.experimental import pallas as pl

N = 4096
H = 256
EPS = 0.3
BM = 512


def _embed_body(f_ref, w_ref, b_ref, out_ref):
    acc = jnp.dot(f_ref[...].astype(jnp.bfloat16), w_ref[...].astype(jnp.bfloat16),
                  preferred_element_type=jnp.float32)
    out_ref[...] = jnp.maximum(acc + b_ref[...], 0.0)


def _embed(feature, lin_w, lin_b):
    n, f_in = feature.shape
    h = lin_w.shape[1]
    return pl.pallas_call(
        _embed_body,
        grid=(n // BM,),
        in_specs=[
            pl.BlockSpec((BM, f_in), lambda i: (i, 0)),
            pl.BlockSpec((f_in, h), lambda i: (0, 0)),
            pl.BlockSpec((1, h), lambda i: (0, 0)),
        ],
        out_specs=pl.BlockSpec((BM, h), lambda i: (i, 0)),
        out_shape=jax.ShapeDtypeStruct((n, h), jnp.float32),
    )(feature, lin_w, lin_b.reshape(1, h))


def _gate_body(h_ref, wgd_ref, wgs_ref, bg_ref, a_ref, bt_ref, hb_ref):
    h = h_ref[...]
    a_ref[...] = jnp.dot(h, wgd_ref[...], preferred_element_type=jnp.float32) + bg_ref[0, 0]
    b = jnp.dot(h, wgs_ref[...], preferred_element_type=jnp.float32)
    bt_ref[...] = b.reshape(1, N)
    hb_ref[...] = h.astype(jnp.bfloat16)


def _gate_proj(h, wg_dst, wg_src, bg):
    return pl.pallas_call(
        _gate_body,
        grid=(1,),
        in_specs=[
            pl.BlockSpec((N, H), lambda i: (0, 0)),
            pl.BlockSpec((H, 1), lambda i: (0, 0)),
            pl.BlockSpec((H, 1), lambda i: (0, 0)),
            pl.BlockSpec((1, 1), lambda i: (0, 0)),
        ],
        out_specs=[
            pl.BlockSpec((N, 1), lambda i: (0, 0)),
            pl.BlockSpec((1, N), lambda i: (0, 0)),
            pl.BlockSpec((N, H), lambda i: (0, 0)),
        ],
        out_shape=[
            jax.ShapeDtypeStruct((N, 1), jnp.float32),
            jax.ShapeDtypeStruct((1, N), jnp.float32),
            jax.ShapeDtypeStruct((N, H), jnp.bfloat16),
        ],
    )(h, wg_dst, wg_src, bg)


def _fa_body(hb_ref, adj_ref, a_ref, bt_ref, h0_ref, out_ref):
    g = jnp.tanh(a_ref[...] + bt_ref[...])
    e = (adj_ref[...] * g).astype(jnp.bfloat16)
    acc = jnp.dot(e, hb_ref[...], preferred_element_type=jnp.float32)
    out_ref[...] = jnp.maximum(acc, 0.0) + EPS * h0_ref[...]


def _fa_layer(hb, adj, a, bt, h0):
    return pl.pallas_call(
        _fa_body,
        grid=(N // BM,),
        in_specs=[
            pl.BlockSpec((N, H), lambda i: (0, 0)),
            pl.BlockSpec((BM, N), lambda i: (i, 0)),
            pl.BlockSpec((BM, 1), lambda i: (i, 0)),
            pl.BlockSpec((1, N), lambda i: (0, 0)),
            pl.BlockSpec((BM, H), lambda i: (i, 0)),
        ],
        out_specs=pl.BlockSpec((BM, H), lambda i: (i, 0)),
        out_shape=jax.ShapeDtypeStruct((N, H), jnp.float32),
    )(hb, adj, a, bt, h0)


def _fc_body(x_ref, w_ref, b_ref, out_ref):
    o = jnp.dot(x_ref[...], w_ref[...], preferred_element_type=jnp.float32)
    o = o + b_ref[...]
    m = jnp.max(o, axis=1, keepdims=True)
    lse = jnp.log(jnp.sum(jnp.exp(o - m), axis=1, keepdims=True))
    out_ref[...] = o - m - lse


def _fc(x, fc_w, fc_b):
    h, c = fc_w.shape
    return pl.pallas_call(
        _fc_body,
        grid=(N // BM,),
        in_specs=[
            pl.BlockSpec((BM, h), lambda i: (i, 0)),
            pl.BlockSpec((h, c), lambda i: (0, 0)),
            pl.BlockSpec((1, c), lambda i: (0, 0)),
        ],
        out_specs=pl.BlockSpec((BM, c), lambda i: (i, 0)),
        out_shape=jax.ShapeDtypeStruct((N, c), jnp.float32),
    )(x, fc_w, fc_b.reshape(1, c))


@jax.jit
def kernel(feature, adj, lin_w, lin_b, gate_w, gate_b, fc_w, fc_b):
    x = _embed(feature, lin_w, lin_b)
    h0 = x
    n_layer = gate_w.shape[0]
    hh = gate_w.shape[1] // 2
    for i in range(n_layer):
        wg_dst = gate_w[i, :hh].reshape(hh, 1)
        wg_src = gate_w[i, hh:].reshape(hh, 1)
        bg = gate_b[i].reshape(1, 1)
        a, bt, hb = _gate_proj(x, wg_dst, wg_src, bg)
        x = _fa_layer(hb, adj, a, bt, h0)
    return _fc(x, fc_w, fc_b)


# speedup vs baseline: 1.3214x; 1.3214x over previous
import jax
import jax.numpy as jnp
from jax.experimental import pallas as pl
from jax.experimental.pallas import tpu as pltpu

N = 4096
F_IN = 512
H = 256
C = 64
EPS = 0.3
NB = 16
BM = N // NB


def _mega_body(feature_ref, adj_ref, lin_wb_ref, lin_b_ref, wg_ref, bg_ref,
               fc_wb_ref, fc_b_ref, out_ref,
               adj_bf, x0, x1, hb1, hb2, a_s, bt_s):
    st = pl.program_id(0)
    i = pl.program_id(1)
    rows = pl.ds(i * BM, BM)

    @pl.when(st == 0)
    def _embed():
        fb = feature_ref[...].astype(jnp.bfloat16)
        acc = jnp.dot(fb, lin_wb_ref[...], preferred_element_type=jnp.float32)
        xb = jnp.maximum(acc + lin_b_ref[...], 0.0)
        x0[rows, :] = xb
        hb1[rows, :] = xb.astype(jnp.bfloat16)

    @pl.when(st == 1)
    def _fa1():
        @pl.when(i == 0)
        def _gate1():
            x = x0[...]
            a_s[...] = jnp.dot(x, wg_ref[:, 0:1],
                               preferred_element_type=jnp.float32) + bg_ref[0, 0]
            b = jnp.dot(x, wg_ref[:, 1:2], preferred_element_type=jnp.float32)
            bt_s[...] = b.reshape(1, N)

        g = jnp.tanh(a_s[rows, :] + bt_s[...])
        adjf = adj_ref[...]
        adj_bf[rows, :] = adjf.astype(jnp.bfloat16)
        e = (adjf * g).astype(jnp.bfloat16)
        acc = jnp.dot(e, hb1[...], preferred_element_type=jnp.float32)
        xb = jnp.maximum(acc, 0.0) + EPS * x0[rows, :]
        x1[rows, :] = xb
        hb2[rows, :] = xb.astype(jnp.bfloat16)

    @pl.when(st == 2)
    def _fa2():
        @pl.when(i == 0)
        def _gate2():
            x = x1[...]
            a_s[...] = jnp.dot(x, wg_ref[:, 2:3],
                               preferred_element_type=jnp.float32) + bg_ref[0, 1]
            b = jnp.dot(x, wg_ref[:, 3:4], preferred_element_type=jnp.float32)
            bt_s[...] = b.reshape(1, N)

        g = jnp.tanh(a_s[rows, :] + bt_s[...])
        adjf = adj_bf[rows, :].astype(jnp.float32)
        e = (adjf * g).astype(jnp.bfloat16)
        acc = jnp.dot(e, hb2[...], preferred_element_type=jnp.float32)
        x2 = jnp.maximum(acc, 0.0) + EPS * x0[rows, :]
        o = jnp.dot(x2.astype(jnp.bfloat16), fc_wb_ref[...],
                    preferred_element_type=jnp.float32) + fc_b_ref[...]
        m = jnp.max(o, axis=1, keepdims=True)
        lse = jnp.log(jnp.sum(jnp.exp(o - m), axis=1, keepdims=True))
        out_ref[...] = o - m - lse


@jax.jit
def kernel(feature, adj, lin_w, lin_b, gate_w, gate_b, fc_w, fc_b):
    hh = gate_w.shape[1] // 2
    wg = jnp.stack([gate_w[0, :hh], gate_w[0, hh:],
                    gate_w[1, :hh], gate_w[1, hh:]], axis=1)
    bg = gate_b.reshape(1, 2)

    return pl.pallas_call(
        _mega_body,
        grid=(3, NB),
        in_specs=[
            pl.BlockSpec((BM, F_IN),
                         lambda st, i: (jnp.where(st == 0, i, NB - 1), 0)),
            pl.BlockSpec((BM, N),
                         lambda st, i: (jnp.where(st == 1, i,
                                                  jnp.where(st == 0, 0, NB - 1)), 0)),
            pl.BlockSpec((F_IN, H), lambda st, i: (0, 0)),
            pl.BlockSpec((1, H), lambda st, i: (0, 0)),
            pl.BlockSpec((H, 4), lambda st, i: (0, 0)),
            pl.BlockSpec((1, 2), lambda st, i: (0, 0)),
            pl.BlockSpec((H, C), lambda st, i: (0, 0)),
            pl.BlockSpec((1, C), lambda st, i: (0, 0)),
        ],
        out_specs=pl.BlockSpec((BM, C), lambda st, i: (i, 0)),
        out_shape=jax.ShapeDtypeStruct((N, C), jnp.float32),
        scratch_shapes=[
            pltpu.VMEM((N, N), jnp.bfloat16),
            pltpu.VMEM((N, H), jnp.float32),
            pltpu.VMEM((N, H), jnp.float32),
            pltpu.VMEM((N, H), jnp.bfloat16),
            pltpu.VMEM((N, H), jnp.bfloat16),
            pltpu.VMEM((N, 1), jnp.float32),
            pltpu.VMEM((1, N), jnp.float32),
        ],
    )(feature, adj, lin_w.astype(jnp.bfloat16), lin_b.reshape(1, H),
      wg, bg, fc_w.astype(jnp.bfloat16), fc_b.reshape(1, C))
